# baseline (speedup 1.0000x reference)
import functools

import jax
import jax.numpy as jnp
from jax import lax
from jax.experimental import pallas as pl
from jax.experimental.pallas import tpu as pltpu
from jax.experimental.pallas import tpu_sc as plsc

N = 10000
D = 128
H = 128
C = 40
E = 320000

NP = 10112
RPT = NP // 16
CH = 128
NW = 32
NCH = 80
NCH2 = NCH // 2
EPAD = NW * NCH * CH
DW = 16

BN = 400
GRID = N // BN

_mesh = plsc.VectorSubcoreMesh(core_axis_name="c", subcore_axis_name="s")


@functools.partial(
    pl.kernel,
    mesh=_mesh,
    out_type=jax.ShapeDtypeStruct((2, NP, H), jnp.float32),
    scratch_types=[
        pltpu.VMEM_SHARED((NP, H), jnp.float32),
        pltpu.VMEM((NCH, CH), jnp.int32),
        pltpu.VMEM((CH, H), jnp.float32),
    ],
)
def _deg_kernel(dsts_hbm, zeros_hbm, ones_hbm, out_hbm, dacc, dst_v, ones_v):
    c = lax.axis_index("c")
    s = lax.axis_index("s")
    wid = s * 2 + c
    row0 = s * RPT
    pltpu.sync_copy(zeros_hbm.at[pl.ds(row0, RPT)], dacc.at[pl.ds(row0, RPT)])
    pltpu.sync_copy(dsts_hbm.at[wid], dst_v)
    pltpu.sync_copy(ones_hbm, ones_v)
    plsc.subcore_barrier()

    def body(j, carry):
        pltpu.sync_copy(ones_v, dacc.at[dst_v.at[j]], add=True)
        return carry

    lax.fori_loop(0, NCH, body, 0)
    plsc.subcore_barrier()
    pltpu.sync_copy(dacc.at[pl.ds(row0, RPT)], out_hbm.at[c].at[pl.ds(row0, RPT)])


@functools.partial(
    pl.kernel,
    mesh=_mesh,
    out_type=jax.ShapeDtypeStruct((2, NP, H), jnp.float32),
    scratch_types=[
        pltpu.VMEM_SHARED((NP, H), jnp.float32),
        pltpu.VMEM((NCH, CH), jnp.int32),
        pltpu.VMEM((NCH, CH), jnp.int32),
        pltpu.VMEM((CH, H), jnp.float32),
        pltpu.SemaphoreType.DMA,
    ],
)
def _agg_kernel(hs_hbm, srcs_hbm, dsts_hbm, zeros_hbm, out_hbm,
                acc, src_v, dst_v, rows_v, sem):
    c = lax.axis_index("c")
    s = lax.axis_index("s")
    wid = s * 2 + c
    row0 = s * RPT
    pltpu.sync_copy(zeros_hbm.at[pl.ds(row0, RPT)], acc.at[pl.ds(row0, RPT)])
    pltpu.sync_copy(srcs_hbm.at[wid], src_v)
    pltpu.sync_copy(dsts_hbm.at[wid], dst_v)
    plsc.subcore_barrier()

    def body(j, carry):
        pltpu.async_copy(hs_hbm.at[src_v.at[j]], rows_v, sem).wait()
        pltpu.sync_copy(rows_v, acc.at[dst_v.at[j]], add=True)
        return carry

    lax.fori_loop(0, NCH, body, 0)
    plsc.subcore_barrier()
    pltpu.sync_copy(acc.at[pl.ds(row0, RPT)], out_hbm.at[c].at[pl.ds(row0, RPT)])


def _mm1_body(deg_ref, x_ref, w_ref, hs_ref, dinv_ref):
    d = deg_ref[...]
    deg = d[0, :, 0] + d[1, :, 0] + 1.0
    dinv = lax.rsqrt(deg)
    h = jnp.dot(x_ref[...], w_ref[...], preferred_element_type=jnp.float32)
    hs_ref[...] = h * dinv[:, None]
    dinv_ref[...] = dinv[:, None]


_mm1 = pl.pallas_call(
    _mm1_body,
    grid=(GRID,),
    in_specs=[
        pl.BlockSpec((2, BN, H), lambda i: (0, i, 0)),
        pl.BlockSpec((BN, D), lambda i: (i, 0)),
        pl.BlockSpec((D, H), lambda i: (0, 0)),
    ],
    out_specs=[
        pl.BlockSpec((BN, H), lambda i: (i, 0)),
        pl.BlockSpec((BN, 1), lambda i: (i, 0)),
    ],
    out_shape=[
        jax.ShapeDtypeStruct((N, H), jnp.float32),
        jax.ShapeDtypeStruct((N, 1), jnp.float32),
    ],
)


def _fuse2_body(p_ref, hs_ref, dinv_ref, b_ref, w_ref, out_ref):
    pp = p_ref[...]
    acc = pp[0] + pp[1] + hs_ref[...]
    dinv = dinv_ref[...]
    t = acc * dinv + b_ref[...]
    r = jnp.maximum(t, 0.0)
    out_ref[...] = jnp.dot(r, w_ref[...], preferred_element_type=jnp.float32) * dinv


_fuse2 = pl.pallas_call(
    _fuse2_body,
    grid=(GRID,),
    in_specs=[
        pl.BlockSpec((2, BN, H), lambda i: (0, i, 0)),
        pl.BlockSpec((BN, H), lambda i: (i, 0)),
        pl.BlockSpec((BN, 1), lambda i: (i, 0)),
        pl.BlockSpec((1, H), lambda i: (0, 0)),
        pl.BlockSpec((H, H), lambda i: (0, 0)),
    ],
    out_specs=pl.BlockSpec((BN, H), lambda i: (i, 0)),
    out_shape=jax.ShapeDtypeStruct((N, H), jnp.float32),
)


def _fuse3_body(p_ref, hs_ref, dinv_ref, b_ref, w_ref, bc_ref, out_ref):
    pp = p_ref[...]
    acc = pp[0] + pp[1] + hs_ref[...]
    t = acc * dinv_ref[...] + b_ref[...]
    out_ref[...] = (
        jnp.dot(t, w_ref[...], preferred_element_type=jnp.float32) + bc_ref[...]
    )


_fuse3 = pl.pallas_call(
    _fuse3_body,
    grid=(GRID,),
    in_specs=[
        pl.BlockSpec((2, BN, H), lambda i: (0, i, 0)),
        pl.BlockSpec((BN, H), lambda i: (i, 0)),
        pl.BlockSpec((BN, 1), lambda i: (i, 0)),
        pl.BlockSpec((1, H), lambda i: (0, 0)),
        pl.BlockSpec((H, C), lambda i: (0, 0)),
        pl.BlockSpec((1, C), lambda i: (0, 0)),
    ],
    out_specs=pl.BlockSpec((BN, C), lambda i: (i, 0)),
    out_shape=jax.ShapeDtypeStruct((N, C), jnp.float32),
)


def kernel(x, edge_index, W1, b1, W2, b2, Wc, bc):
    src = edge_index[0]
    dst = edge_index[1]
    pad = EPAD - E
    srcp = jnp.concatenate([src, jnp.zeros((pad,), jnp.int32)]).reshape(NW, NCH, CH)
    dstp = jnp.concatenate([dst, jnp.full((pad,), N, jnp.int32)]).reshape(NW, NCH, CH)
    zeros_acc = jnp.zeros((NP, H), jnp.float32)
    ones_blk = jnp.ones((CH, H), jnp.float32)

    degp = _deg_kernel(dstp, zeros_acc, ones_blk)
    hs1, dinv = _mm1(degp, x, W1)
    p1 = _agg_kernel(hs1, srcp, dstp, zeros_acc)
    hs2 = _fuse2(p1, hs1, dinv, b1.reshape(1, H), W2)
    p2 = _agg_kernel(hs2, srcp, dstp, zeros_acc)
    logits = _fuse3(p2, hs2, dinv, b2.reshape(1, H), Wc, bc.reshape(1, C))
    return logits


# speedup vs baseline: 2.3928x; 2.3928x over previous
import functools

import jax
import jax.numpy as jnp
from jax import lax
from jax.experimental import pallas as pl
from jax.experimental.pallas import tpu as pltpu
from jax.experimental.pallas import tpu_sc as plsc

N = 10000
D = 128
H = 128
C = 40
E = 320000

NP = 10112
RPT = NP // 16
CH = 128
NW = 32
NCH = 80
NCH2 = NCH // 2
EPAD = NW * NCH * CH
DW = 16

BN = 400
GRID = N // BN

_mesh = plsc.VectorSubcoreMesh(core_axis_name="c", subcore_axis_name="s")


@functools.partial(
    pl.kernel,
    mesh=_mesh,
    out_type=jax.ShapeDtypeStruct((2, NP, H), jnp.float32),
    scratch_types=[
        pltpu.VMEM_SHARED((NP, H), jnp.float32),
        pltpu.VMEM((NCH, CH), jnp.int32),
        pltpu.VMEM((CH, H), jnp.float32),
    ],
)
def _deg_kernel(dsts_hbm, zeros_hbm, ones_hbm, out_hbm, dacc, dst_v, ones_v):
    c = lax.axis_index("c")
    s = lax.axis_index("s")
    wid = s * 2 + c
    row0 = s * RPT
    pltpu.sync_copy(zeros_hbm.at[pl.ds(row0, RPT)], dacc.at[pl.ds(row0, RPT)])
    pltpu.sync_copy(dsts_hbm.at[wid], dst_v)
    pltpu.sync_copy(ones_hbm, ones_v)
    plsc.subcore_barrier()

    def body(j, carry):
        pltpu.sync_copy(ones_v, dacc.at[dst_v.at[j]], add=True)
        return carry

    lax.fori_loop(0, NCH, body, 0)
    plsc.subcore_barrier()
    pltpu.sync_copy(dacc.at[pl.ds(row0, RPT)], out_hbm.at[c].at[pl.ds(row0, RPT)])


@functools.partial(
    pl.kernel,
    mesh=_mesh,
    out_type=jax.ShapeDtypeStruct((2, NP, H), jnp.float32),
    scratch_types=[
        pltpu.VMEM_SHARED((NP, H), jnp.float32),
        pltpu.VMEM((NCH, CH), jnp.int32),
        pltpu.VMEM((NCH, CH), jnp.int32),
        pltpu.VMEM((CH, H), jnp.float32),
        pltpu.SemaphoreType.DMA,
    ],
)
def _agg_kernel(hs_hbm, srcs_hbm, dsts_hbm, zeros_hbm, out_hbm,
                acc, src_v, dst_v, rows_v, sem):
    c = lax.axis_index("c")
    s = lax.axis_index("s")
    wid = s * 2 + c
    row0 = s * RPT
    pltpu.sync_copy(zeros_hbm.at[pl.ds(row0, RPT)], acc.at[pl.ds(row0, RPT)])
    pltpu.sync_copy(srcs_hbm.at[wid], src_v)
    pltpu.sync_copy(dsts_hbm.at[wid], dst_v)
    plsc.subcore_barrier()

    def body(j, carry):
        pltpu.async_copy(hs_hbm.at[src_v.at[j]], rows_v, sem).wait()
        pltpu.sync_copy(rows_v, acc.at[dst_v.at[j]], add=True)
        return carry

    lax.fori_loop(0, NCH, body, 0)
    plsc.subcore_barrier()
    pltpu.sync_copy(acc.at[pl.ds(row0, RPT)], out_hbm.at[c].at[pl.ds(row0, RPT)])


def _mm1_body(deg_ref, x_ref, w_ref, hs_ref, dinv_ref):
    d = deg_ref[...]
    deg = d[0, :, 0] + d[1, :, 0] + 1.0
    dinv = lax.rsqrt(deg)
    h = jnp.dot(x_ref[...], w_ref[...], preferred_element_type=jnp.float32)
    hs_ref[...] = h * dinv[:, None]
    dinv_ref[...] = dinv[:, None]


_mm1 = pl.pallas_call(
    _mm1_body,
    grid=(GRID,),
    in_specs=[
        pl.BlockSpec((2, BN, H), lambda i: (0, i, 0)),
        pl.BlockSpec((BN, D), lambda i: (i, 0)),
        pl.BlockSpec((D, H), lambda i: (0, 0)),
    ],
    out_specs=[
        pl.BlockSpec((BN, H), lambda i: (i, 0)),
        pl.BlockSpec((BN, 1), lambda i: (i, 0)),
    ],
    out_shape=[
        jax.ShapeDtypeStruct((N, H), jnp.float32),
        jax.ShapeDtypeStruct((N, 1), jnp.float32),
    ],
)


def _fuse2_body(p_ref, hs_ref, dinv_ref, b_ref, w_ref, out_ref):
    pp = p_ref[...]
    acc = pp[0] + pp[1] + hs_ref[...]
    dinv = dinv_ref[...]
    t = acc * dinv + b_ref[...]
    r = jnp.maximum(t, 0.0)
    out_ref[...] = jnp.dot(r, w_ref[...], preferred_element_type=jnp.float32) * dinv


_fuse2 = pl.pallas_call(
    _fuse2_body,
    grid=(GRID,),
    in_specs=[
        pl.BlockSpec((2, BN, H), lambda i: (0, i, 0)),
        pl.BlockSpec((BN, H), lambda i: (i, 0)),
        pl.BlockSpec((BN, 1), lambda i: (i, 0)),
        pl.BlockSpec((1, H), lambda i: (0, 0)),
        pl.BlockSpec((H, H), lambda i: (0, 0)),
    ],
    out_specs=pl.BlockSpec((BN, H), lambda i: (i, 0)),
    out_shape=jax.ShapeDtypeStruct((N, H), jnp.float32),
)


def _fuse3_body(p_ref, hs_ref, dinv_ref, b_ref, w_ref, bc_ref, out_ref):
    pp = p_ref[...]
    acc = pp[0] + pp[1] + hs_ref[...]
    t = acc * dinv_ref[...] + b_ref[...]
    out_ref[...] = (
        jnp.dot(t, w_ref[...], preferred_element_type=jnp.float32) + bc_ref[...]
    )


_fuse3 = pl.pallas_call(
    _fuse3_body,
    grid=(GRID,),
    in_specs=[
        pl.BlockSpec((2, BN, H), lambda i: (0, i, 0)),
        pl.BlockSpec((BN, H), lambda i: (i, 0)),
        pl.BlockSpec((BN, 1), lambda i: (i, 0)),
        pl.BlockSpec((1, H), lambda i: (0, 0)),
        pl.BlockSpec((H, C), lambda i: (0, 0)),
        pl.BlockSpec((1, C), lambda i: (0, 0)),
    ],
    out_specs=pl.BlockSpec((BN, C), lambda i: (i, 0)),
    out_shape=jax.ShapeDtypeStruct((N, C), jnp.float32),
)


def kernel(x, edge_index, W1, b1, W2, b2, Wc, bc):
    src = edge_index[0]
    dst = edge_index[1]
    pad = EPAD - E
    pad_src = jnp.arange(pad, dtype=jnp.int32) % N
    pad_dst = N + jnp.arange(pad, dtype=jnp.int32) % (NP - N)
    srcp = jnp.concatenate([src, pad_src]).reshape(NW, NCH, CH)
    dstp = jnp.concatenate([dst, pad_dst]).reshape(NW, NCH, CH)
    zeros_acc = jnp.zeros((NP, H), jnp.float32)
    ones_blk = jnp.ones((CH, H), jnp.float32)

    degp = _deg_kernel(dstp, zeros_acc, ones_blk)
    hs1, dinv = _mm1(degp, x, W1)
    p1 = _agg_kernel(hs1, srcp, dstp, zeros_acc)
    hs2 = _fuse2(p1, hs1, dinv, b1.reshape(1, H), W2)
    p2 = _agg_kernel(hs2, srcp, dstp, zeros_acc)
    logits = _fuse3(p2, hs2, dinv, b2.reshape(1, H), Wc, bc.reshape(1, C))
    return logits


# speedup vs baseline: 2.7005x; 1.1286x over previous
import functools

import jax
import jax.numpy as jnp
from jax import lax
from jax.experimental import pallas as pl
from jax.experimental.pallas import tpu as pltpu
from jax.experimental.pallas import tpu_sc as plsc

N = 10000
D = 128
H = 128
C = 40
E = 320000

NP = 10112
RPT = NP // 16
CH = 128
NW = 32
NCH = 80
NCH2 = NCH // 2
EPAD = NW * NCH * CH
DW = 16

BN = 400
GRID = N // BN

_mesh = plsc.VectorSubcoreMesh(core_axis_name="c", subcore_axis_name="s")


@functools.partial(
    pl.kernel,
    mesh=_mesh,
    out_type=jax.ShapeDtypeStruct((2, NP, H), jnp.float32),
    scratch_types=[
        pltpu.VMEM_SHARED((NP, H), jnp.float32),
        pltpu.VMEM((NCH, CH), jnp.int32),
        pltpu.VMEM((CH, H), jnp.float32),
    ],
)
def _deg_kernel(dsts_hbm, zeros_hbm, ones_hbm, out_hbm, dacc, dst_v, ones_v):
    c = lax.axis_index("c")
    s = lax.axis_index("s")
    wid = s * 2 + c
    row0 = s * RPT
    pltpu.sync_copy(zeros_hbm.at[pl.ds(row0, RPT)], dacc.at[pl.ds(row0, RPT)])
    pltpu.sync_copy(dsts_hbm.at[wid], dst_v)
    pltpu.sync_copy(ones_hbm, ones_v)
    plsc.subcore_barrier()

    def body(j, carry):
        pltpu.sync_copy(ones_v, dacc.at[dst_v.at[j]], add=True)
        return carry

    lax.fori_loop(0, NCH, body, 0)
    plsc.subcore_barrier()
    pltpu.sync_copy(dacc.at[pl.ds(row0, RPT)], out_hbm.at[c].at[pl.ds(row0, RPT)])


@functools.partial(
    pl.kernel,
    mesh=_mesh,
    out_type=jax.ShapeDtypeStruct((2, NP, H), jnp.float32),
    scratch_types=[
        pltpu.VMEM_SHARED((NP, H), jnp.float32),
        pltpu.VMEM((NCH // 2, CH), jnp.int32),
        pltpu.VMEM((NCH // 2, CH), jnp.int32),
        pltpu.VMEM((2, CH), jnp.int32),
        pltpu.VMEM((2, CH), jnp.int32),
        pltpu.VMEM((CH, H), jnp.float32),
        pltpu.VMEM((CH, H), jnp.float32),
        pltpu.SemaphoreType.DMA,
        pltpu.SemaphoreType.DMA,
    ],
)
def _agg_kernel(hs_hbm, srcs_hbm, dsts_hbm, zeros_hbm, out_hbm,
                acc, src16_v, dst16_v, src_st, dst_st, rows_a, rows_b,
                sem_a, sem_b):
    c = lax.axis_index("c")
    s = lax.axis_index("s")
    wid = s * 2 + c
    row0 = s * RPT
    pltpu.sync_copy(zeros_hbm.at[pl.ds(row0, RPT)], acc.at[pl.ds(row0, RPT)])
    pltpu.sync_copy(srcs_hbm.at[wid], src16_v)
    pltpu.sync_copy(dsts_hbm.at[wid], dst16_v)
    plsc.subcore_barrier()

    def cvt(r, half, p):
        for g in range(4):
            sw = src16_v[r, pl.ds(half * 64 + g * 16, 16)]
            dw = dst16_v[r, pl.ds(half * 64 + g * 16, 16)]
            src_st[p, pl.ds(g * 32, 16)] = jnp.bitwise_and(sw, 0xFFFF)
            src_st[p, pl.ds(g * 32 + 16, 16)] = lax.shift_right_logical(sw, 16)
            dst_st[p, pl.ds(g * 32, 16)] = jnp.bitwise_and(dw, 0xFFFF)
            dst_st[p, pl.ds(g * 32 + 16, 16)] = lax.shift_right_logical(dw, 16)

    cvt(0, 0, 0)
    pltpu.async_copy(hs_hbm.at[src_st.at[0]], rows_a, sem_a)

    def body(jj, carry):
        cvt(jj, 1, 1)
        pltpu.make_async_copy(hs_hbm.at[src_st.at[0]], rows_a, sem_a).wait()
        pltpu.async_copy(hs_hbm.at[src_st.at[1]], rows_b, sem_b)
        pltpu.sync_copy(rows_a, acc.at[dst_st.at[0]], add=True)

        @pl.when(jj + 1 < NCH2)
        def _next_even():
            cvt(jj + 1, 0, 0)

        pltpu.make_async_copy(hs_hbm.at[src_st.at[1]], rows_b, sem_b).wait()

        @pl.when(jj + 1 < NCH2)
        def _issue():
            pltpu.async_copy(hs_hbm.at[src_st.at[0]], rows_a, sem_a)

        pltpu.sync_copy(rows_b, acc.at[dst_st.at[1]], add=True)
        return carry

    lax.fori_loop(0, NCH2, body, 0)
    plsc.subcore_barrier()
    pltpu.sync_copy(acc.at[pl.ds(row0, RPT)], out_hbm.at[c].at[pl.ds(row0, RPT)])


def _mm1_body(deg_ref, x_ref, w_ref, hs_ref, dinv_ref):
    d = deg_ref[...]
    deg = d[0, :, 0] + d[1, :, 0] + 1.0
    dinv = lax.rsqrt(deg)
    h = jnp.dot(x_ref[...], w_ref[...], preferred_element_type=jnp.float32)
    hs_ref[...] = h * dinv[:, None]
    dinv_ref[...] = dinv[:, None]


_mm1 = pl.pallas_call(
    _mm1_body,
    grid=(GRID,),
    in_specs=[
        pl.BlockSpec((2, BN, H), lambda i: (0, i, 0)),
        pl.BlockSpec((BN, D), lambda i: (i, 0)),
        pl.BlockSpec((D, H), lambda i: (0, 0)),
    ],
    out_specs=[
        pl.BlockSpec((BN, H), lambda i: (i, 0)),
        pl.BlockSpec((BN, 1), lambda i: (i, 0)),
    ],
    out_shape=[
        jax.ShapeDtypeStruct((N, H), jnp.float32),
        jax.ShapeDtypeStruct((N, 1), jnp.float32),
    ],
)


def _fuse2_body(p_ref, hs_ref, dinv_ref, b_ref, w_ref, out_ref):
    pp = p_ref[...]
    acc = pp[0] + pp[1] + hs_ref[...]
    dinv = dinv_ref[...]
    t = acc * dinv + b_ref[...]
    r = jnp.maximum(t, 0.0)
    out_ref[...] = jnp.dot(r, w_ref[...], preferred_element_type=jnp.float32) * dinv


_fuse2 = pl.pallas_call(
    _fuse2_body,
    grid=(GRID,),
    in_specs=[
        pl.BlockSpec((2, BN, H), lambda i: (0, i, 0)),
        pl.BlockSpec((BN, H), lambda i: (i, 0)),
        pl.BlockSpec((BN, 1), lambda i: (i, 0)),
        pl.BlockSpec((1, H), lambda i: (0, 0)),
        pl.BlockSpec((H, H), lambda i: (0, 0)),
    ],
    out_specs=pl.BlockSpec((BN, H), lambda i: (i, 0)),
    out_shape=jax.ShapeDtypeStruct((N, H), jnp.float32),
)


def _fuse3_body(p_ref, hs_ref, dinv_ref, b_ref, w_ref, bc_ref, out_ref):
    pp = p_ref[...]
    acc = pp[0] + pp[1] + hs_ref[...]
    t = acc * dinv_ref[...] + b_ref[...]
    out_ref[...] = (
        jnp.dot(t, w_ref[...], preferred_element_type=jnp.float32) + bc_ref[...]
    )


_fuse3 = pl.pallas_call(
    _fuse3_body,
    grid=(GRID,),
    in_specs=[
        pl.BlockSpec((2, BN, H), lambda i: (0, i, 0)),
        pl.BlockSpec((BN, H), lambda i: (i, 0)),
        pl.BlockSpec((BN, 1), lambda i: (i, 0)),
        pl.BlockSpec((1, H), lambda i: (0, 0)),
        pl.BlockSpec((H, C), lambda i: (0, 0)),
        pl.BlockSpec((1, C), lambda i: (0, 0)),
    ],
    out_specs=pl.BlockSpec((BN, C), lambda i: (i, 0)),
    out_shape=jax.ShapeDtypeStruct((N, C), jnp.float32),
)


def kernel(x, edge_index, W1, b1, W2, b2, Wc, bc):
    src = edge_index[0]
    dst = edge_index[1]
    pad = EPAD - E
    pad_src = jnp.arange(pad, dtype=jnp.int32) % N
    pad_dst = N + jnp.arange(pad, dtype=jnp.int32) % (NP - N)
    srcp = jnp.concatenate([src, pad_src]).reshape(NW, NCH, CH)
    dstp = jnp.concatenate([dst, pad_dst]).reshape(NW, NCH, CH)
    srcp16 = (srcp[:, :, 0::2] | (srcp[:, :, 1::2] << 16)).reshape(NW, NCH // 2, CH)
    dstp16 = (dstp[:, :, 0::2] | (dstp[:, :, 1::2] << 16)).reshape(NW, NCH // 2, CH)
    zeros_acc = jnp.zeros((NP, H), jnp.float32)
    ones_blk = jnp.ones((CH, H), jnp.float32)

    degp = _deg_kernel(dstp, zeros_acc, ones_blk)
    hs1, dinv = _mm1(degp, x, W1)
    p1 = _agg_kernel(hs1, srcp16, dstp16, zeros_acc)
    hs2 = _fuse2(p1, hs1, dinv, b1.reshape(1, H), W2)
    p2 = _agg_kernel(hs2, srcp16, dstp16, zeros_acc)
    logits = _fuse3(p2, hs2, dinv, b2.reshape(1, H), Wc, bc.reshape(1, C))
    return logits


# speedup vs baseline: 2.7079x; 1.0027x over previous
import functools

import jax
import jax.numpy as jnp
from jax import lax
from jax.experimental import pallas as pl
from jax.experimental.pallas import tpu as pltpu
from jax.experimental.pallas import tpu_sc as plsc

N = 10000
D = 128
H = 128
C = 40
E = 320000

NP = 10112
RPT = NP // 16
CH = 128
NW = 32
NCH = 80
NCH2 = NCH // 2
EPAD = NW * NCH * CH
DW = 16

BN = 400
GRID = N // BN

_mesh = plsc.VectorSubcoreMesh(core_axis_name="c", subcore_axis_name="s")


@functools.partial(
    pl.kernel,
    mesh=_mesh,
    out_type=jax.ShapeDtypeStruct((2, NP, H), jnp.float32),
    scratch_types=[
        pltpu.VMEM_SHARED((NP, H), jnp.float32),
        pltpu.VMEM((NCH, CH), jnp.int32),
        pltpu.VMEM((CH, H), jnp.float32),
        pltpu.SemaphoreType.DMA,
    ],
)
def _deg_kernel(dsts_hbm, zeros_hbm, ones_hbm, out_hbm, dacc, dst_v, ones_v, sem):
    c = lax.axis_index("c")
    s = lax.axis_index("s")
    wid = s * 2 + c
    row0 = s * RPT
    pltpu.sync_copy(zeros_hbm.at[pl.ds(row0, RPT)], dacc.at[pl.ds(row0, RPT)])
    pltpu.sync_copy(dsts_hbm.at[wid], dst_v)
    pltpu.sync_copy(ones_hbm, ones_v)
    plsc.subcore_barrier()

    def body(g, carry):
        for k in range(8):
            pltpu.async_copy(ones_v, dacc.at[dst_v.at[g * 8 + k]], sem, add=True)
        for k in range(8):
            pltpu.make_async_copy(ones_v, dacc.at[dst_v.at[g * 8 + k]], sem).wait()
        return carry

    lax.fori_loop(0, NCH // 8, body, 0)
    plsc.subcore_barrier()
    pltpu.sync_copy(dacc.at[pl.ds(row0, RPT)], out_hbm.at[c].at[pl.ds(row0, RPT)])


@functools.partial(
    pl.kernel,
    mesh=_mesh,
    out_type=jax.ShapeDtypeStruct((2, NP, H), jnp.float32),
    scratch_types=[
        pltpu.VMEM_SHARED((NP, H), jnp.float32),
        pltpu.VMEM((NCH // 2, CH), jnp.int32),
        pltpu.VMEM((NCH // 2, CH), jnp.int32),
        pltpu.VMEM((2, CH), jnp.int32),
        pltpu.VMEM((2, CH), jnp.int32),
        pltpu.VMEM((CH, H), jnp.float32),
        pltpu.VMEM((CH, H), jnp.float32),
        pltpu.SemaphoreType.DMA,
        pltpu.SemaphoreType.DMA,
    ],
)
def _agg_kernel(hs_hbm, srcs_hbm, dsts_hbm, zeros_hbm, out_hbm,
                acc, src16_v, dst16_v, src_st, dst_st, rows_a, rows_b,
                sem_a, sem_b):
    c = lax.axis_index("c")
    s = lax.axis_index("s")
    wid = s * 2 + c
    row0 = s * RPT
    pltpu.sync_copy(zeros_hbm.at[pl.ds(row0, RPT)], acc.at[pl.ds(row0, RPT)])
    pltpu.sync_copy(srcs_hbm.at[wid], src16_v)
    pltpu.sync_copy(dsts_hbm.at[wid], dst16_v)
    plsc.subcore_barrier()

    def cvt(r, half, p):
        for g in range(4):
            sw = src16_v[r, pl.ds(half * 64 + g * 16, 16)]
            dw = dst16_v[r, pl.ds(half * 64 + g * 16, 16)]
            src_st[p, pl.ds(g * 32, 16)] = jnp.bitwise_and(sw, 0xFFFF)
            src_st[p, pl.ds(g * 32 + 16, 16)] = lax.shift_right_logical(sw, 16)
            dst_st[p, pl.ds(g * 32, 16)] = jnp.bitwise_and(dw, 0xFFFF)
            dst_st[p, pl.ds(g * 32 + 16, 16)] = lax.shift_right_logical(dw, 16)

    cvt(0, 0, 0)
    pltpu.async_copy(hs_hbm.at[src_st.at[0]], rows_a, sem_a)

    def body(jj, carry):
        cvt(jj, 1, 1)
        pltpu.make_async_copy(hs_hbm.at[src_st.at[0]], rows_a, sem_a).wait()
        pltpu.async_copy(hs_hbm.at[src_st.at[1]], rows_b, sem_b)
        pltpu.sync_copy(rows_a, acc.at[dst_st.at[0]], add=True)

        @pl.when(jj + 1 < NCH2)
        def _next_even():
            cvt(jj + 1, 0, 0)

        pltpu.make_async_copy(hs_hbm.at[src_st.at[1]], rows_b, sem_b).wait()

        @pl.when(jj + 1 < NCH2)
        def _issue():
            pltpu.async_copy(hs_hbm.at[src_st.at[0]], rows_a, sem_a)

        pltpu.sync_copy(rows_b, acc.at[dst_st.at[1]], add=True)
        return carry

    lax.fori_loop(0, NCH2, body, 0)
    plsc.subcore_barrier()
    pltpu.sync_copy(acc.at[pl.ds(row0, RPT)], out_hbm.at[c].at[pl.ds(row0, RPT)])


def _mm1_body(deg_ref, x_ref, w_ref, hs_ref, dinv_ref):
    d = deg_ref[...]
    deg = d[0, :, 0] + d[1, :, 0] + 1.0
    dinv = lax.rsqrt(deg)
    h = jnp.dot(x_ref[...], w_ref[...], preferred_element_type=jnp.float32)
    hs_ref[...] = h * dinv[:, None]
    dinv_ref[...] = dinv[:, None]


_mm1 = pl.pallas_call(
    _mm1_body,
    grid=(GRID,),
    in_specs=[
        pl.BlockSpec((2, BN, H), lambda i: (0, i, 0)),
        pl.BlockSpec((BN, D), lambda i: (i, 0)),
        pl.BlockSpec((D, H), lambda i: (0, 0)),
    ],
    out_specs=[
        pl.BlockSpec((BN, H), lambda i: (i, 0)),
        pl.BlockSpec((BN, 1), lambda i: (i, 0)),
    ],
    out_shape=[
        jax.ShapeDtypeStruct((N, H), jnp.float32),
        jax.ShapeDtypeStruct((N, 1), jnp.float32),
    ],
)


def _fuse2_body(p_ref, hs_ref, dinv_ref, b_ref, w_ref, out_ref):
    pp = p_ref[...]
    acc = pp[0] + pp[1] + hs_ref[...]
    dinv = dinv_ref[...]
    t = acc * dinv + b_ref[...]
    r = jnp.maximum(t, 0.0)
    out_ref[...] = jnp.dot(r, w_ref[...], preferred_element_type=jnp.float32) * dinv


_fuse2 = pl.pallas_call(
    _fuse2_body,
    grid=(GRID,),
    in_specs=[
        pl.BlockSpec((2, BN, H), lambda i: (0, i, 0)),
        pl.BlockSpec((BN, H), lambda i: (i, 0)),
        pl.BlockSpec((BN, 1), lambda i: (i, 0)),
        pl.BlockSpec((1, H), lambda i: (0, 0)),
        pl.BlockSpec((H, H), lambda i: (0, 0)),
    ],
    out_specs=pl.BlockSpec((BN, H), lambda i: (i, 0)),
    out_shape=jax.ShapeDtypeStruct((N, H), jnp.float32),
)


def _fuse3_body(p_ref, hs_ref, dinv_ref, b_ref, w_ref, bc_ref, out_ref):
    pp = p_ref[...]
    acc = pp[0] + pp[1] + hs_ref[...]
    t = acc * dinv_ref[...] + b_ref[...]
    out_ref[...] = (
        jnp.dot(t, w_ref[...], preferred_element_type=jnp.float32) + bc_ref[...]
    )


_fuse3 = pl.pallas_call(
    _fuse3_body,
    grid=(GRID,),
    in_specs=[
        pl.BlockSpec((2, BN, H), lambda i: (0, i, 0)),
        pl.BlockSpec((BN, H), lambda i: (i, 0)),
        pl.BlockSpec((BN, 1), lambda i: (i, 0)),
        pl.BlockSpec((1, H), lambda i: (0, 0)),
        pl.BlockSpec((H, C), lambda i: (0, 0)),
        pl.BlockSpec((1, C), lambda i: (0, 0)),
    ],
    out_specs=pl.BlockSpec((BN, C), lambda i: (i, 0)),
    out_shape=jax.ShapeDtypeStruct((N, C), jnp.float32),
)


def kernel(x, edge_index, W1, b1, W2, b2, Wc, bc):
    src = edge_index[0]
    dst = edge_index[1]
    pad = EPAD - E
    pad_src = jnp.arange(pad, dtype=jnp.int32) % N
    pad_dst = N + jnp.arange(pad, dtype=jnp.int32) % (NP - N)
    srcp = jnp.concatenate([src, pad_src]).reshape(NW, NCH, CH)
    dstp = jnp.concatenate([dst, pad_dst]).reshape(NW, NCH, CH)
    srcp16 = (srcp[:, :, 0::2] | (srcp[:, :, 1::2] << 16)).reshape(NW, NCH // 2, CH)
    dstp16 = (dstp[:, :, 0::2] | (dstp[:, :, 1::2] << 16)).reshape(NW, NCH // 2, CH)
    zeros_acc = jnp.zeros((NP, H), jnp.float32)
    ones_blk = jnp.ones((CH, H), jnp.float32)

    degp = _deg_kernel(dstp, zeros_acc, ones_blk)
    hs1, dinv = _mm1(degp, x, W1)
    p1 = _agg_kernel(hs1, srcp16, dstp16, zeros_acc)
    hs2 = _fuse2(p1, hs1, dinv, b1.reshape(1, H), W2)
    p2 = _agg_kernel(hs2, srcp16, dstp16, zeros_acc)
    logits = _fuse3(p2, hs2, dinv, b2.reshape(1, H), Wc, bc.reshape(1, C))
    return logits
